# baseline (speedup 1.0000x reference)
import functools

import jax
import jax.numpy as jnp
import numpy as np
from jax import lax
from jax.experimental import pallas as pl

_N = 128
_r, _c = np.triu_indices(_N)
_TRIU = np.stack([_r, _c], axis=1).reshape(-1).astype(np.int32)
_M = _TRIU.shape[0]

_A_NP = np.zeros((_N, _M), dtype=np.float32)
_A_NP[_TRIU, np.arange(_M)] = 1.0

_BS1 = 8
_BS2 = 256
_MS2 = 5504


def _perm_body(x_ref, p_ref):
    for b in range(_BS1):
        x = x_ref[0, b]
        xt = x.T
        y = xt * xt
        acc = y[0:8]
        for e in range(1, 16):
            acc = acc + y[8 * e:8 * e + 8]
        a4 = acc[0:4] + acc[4:8]
        a2 = a4[0:2] + a4[2:4]
        s = a2[0] + a2[1]
        n = jnp.sqrt(s)
        ni = n[:, None]
        nj = n[None, :]
        ii = lax.broadcasted_iota(jnp.int32, (_N, _N), 0)
        jj = lax.broadcasted_iota(jnp.int32, (_N, _N), 1)
        cmp = (nj > ni) | ((nj == ni) & (jj < ii))
        rank = jnp.sum(cmp.astype(jnp.int32), axis=1)
        top = jnp.sum(jnp.where((rank == 0)[None, :], xt, 0.0), axis=1)
        cc = lax.broadcasted_iota(jnp.int32, (_N, _N), 0)
        oh = rank[None, :] == cc
        p_ref[0, b] = jnp.sum(jnp.where(oh, top[None, :], 0.0), axis=1)


def _expand_body(p_ref, a_ref, o_ref):
    o_ref[...] = jnp.dot(p_ref[...], a_ref[...],
                         preferred_element_type=jnp.float32)


@jax.jit
def kernel(X, noise):
    del noise
    B = X.shape[0]
    X4 = X.reshape(B // _BS1, _BS1, _N, _N)
    P = pl.pallas_call(
        _perm_body,
        grid=(B // _BS1,),
        in_specs=[pl.BlockSpec((1, _BS1, _N, _N), lambda b: (b, 0, 0, 0))],
        out_specs=pl.BlockSpec((1, _BS1, _N), lambda b: (b, 0, 0)),
        out_shape=jax.ShapeDtypeStruct((B // _BS1, _BS1, _N), jnp.float32),
    )(X4).reshape(B, _N)
    A = jnp.asarray(_A_NP)
    out = pl.pallas_call(
        _expand_body,
        grid=(B // _BS2, _M // _MS2),
        in_specs=[
            pl.BlockSpec((_BS2, _N), lambda b, m: (b, 0)),
            pl.BlockSpec((_N, _MS2), lambda b, m: (0, m)),
        ],
        out_specs=pl.BlockSpec((_BS2, _MS2), lambda b, m: (b, m)),
        out_shape=jax.ShapeDtypeStruct((B, _M), jnp.float32),
    )(P, A)
    return out


# speedup vs baseline: 3.2374x; 3.2374x over previous
import functools

import jax
import jax.numpy as jnp
import numpy as np
from jax import lax
from jax.experimental import pallas as pl
from jax.experimental.pallas import tpu as pltpu
from jax.experimental.pallas import tpu_sc as plsc

_N = 128
_r, _c = np.triu_indices(_N)
_TRIU = np.stack([_r, _c], axis=1).reshape(-1).astype(np.int32)
_M = _TRIU.shape[0]

_BS1 = 4
_B = 1024
_NW = 32
_SPW = _B // _NW
_L = 16


def _rank_body(xt_ref, r_ref):
    xt = xt_ref[0]
    y = xt * xt
    acc = y[:, 0:8, :]
    for e in range(1, 16):
        acc = acc + y[:, 8 * e:8 * e + 8, :]
    a4 = acc[:, 0:4, :] + acc[:, 4:8, :]
    a2 = a4[:, 0:2, :] + a4[:, 2:4, :]
    s = a2[:, 0, :] + a2[:, 1, :]
    n = jnp.sqrt(s)
    ni = n[:, :, None]
    nj = n[:, None, :]
    ii = lax.broadcasted_iota(jnp.int32, (_BS1, _N, _N), 1)
    jj = lax.broadcasted_iota(jnp.int32, (_BS1, _N, _N), 2)
    cmp = (nj > ni) | ((nj == ni) & (jj < ii))
    rank = jnp.sum(cmp.astype(jnp.int32), axis=2)
    oh = rank[:, None, :] == ii
    r_ref[0] = jnp.sum(jnp.where(oh, jj, 0), axis=2)


def _sc_gather(xf_hbm, rank_hbm, tri_hbm, base_hbm, out_hbm,
               tri_v, rank_v, idx_v, base_v, rows_v, tab_v, out_v,
               sem):
    wid = lax.axis_index("s") * 2 + lax.axis_index("c")
    s0 = wid * _SPW
    pltpu.sync_copy(tri_hbm, tri_v)
    pltpu.sync_copy(rank_hbm.at[pl.ds(s0 * _N, _SPW * _N)], rank_v)
    pltpu.sync_copy(base_hbm.at[pl.ds(s0, _SPW)], base_v)
    iota = lax.iota(jnp.int32, _L)
    for t in range(2):
        g16 = iota + 16 * t
        i016 = plsc.load_gather(rank_v, [g16 * _N])
        b16 = base_v[pl.ds(16 * t, _L)]
        idx_v[pl.ds(16 * t, _L)] = i016 + b16
    pltpu.async_copy(xf_hbm.at[idx_v], rows_v, sem).wait()
    for t in range(_SPW // 2):
        for b in range(2):
            g = 2 * t + b
            gfull = jnp.full((_L,), g, jnp.int32)
            for k in range(8):
                inv16 = rank_v[pl.ds(g * _N + 16 * k, _L)]
                vals = plsc.load_gather(rows_v, [gfull, inv16])
                tab_v[pl.ds(b * _N + 16 * k, _L)] = vals

        def exp_body(j, _):
            for u in range(4):
                jj = j * 4 + u
                t16 = tri_v[pl.ds(16 * jj, _L)]
                out_v[pl.ds(16 * jj, _L)] = plsc.load_gather(tab_v, [t16])
                out_v[pl.ds(_M + 16 * jj, _L)] = plsc.load_gather(
                    tab_v, [t16 + _N])
            return 0

        lax.fori_loop(0, _M // _L // 4, exp_body, 0)
        pltpu.sync_copy(out_v, out_hbm.at[pl.ds((s0 + 2 * t) * _M, 2 * _M)])


@jax.jit
def kernel(X, noise):
    del noise
    B = X.shape[0]
    XT = jnp.swapaxes(X, 1, 2)
    X4 = XT.reshape(B // _BS1, _BS1, _N, _N)
    rank = pl.pallas_call(
        _rank_body,
        grid=(B // _BS1,),
        in_specs=[pl.BlockSpec((1, _BS1, _N, _N), lambda b: (b, 0, 0, 0))],
        out_specs=pl.BlockSpec((1, _BS1, _N), lambda b: (b, 0, 0)),
        out_shape=jax.ShapeDtypeStruct((B // _BS1, _BS1, _N), jnp.int32),
    )(X4).reshape(B * _N)
    Xf = X.reshape(B * _N, _N)
    tri = jnp.asarray(_TRIU)
    base = jnp.arange(B, dtype=jnp.int32) * _N
    mesh = plsc.VectorSubcoreMesh(core_axis_name="c", subcore_axis_name="s")
    sc = functools.partial(
        pl.kernel, mesh=mesh,
        compiler_params=pltpu.CompilerParams(needs_layout_passes=False),
        out_type=jax.ShapeDtypeStruct((B * _M,), jnp.float32),
        scratch_types=[
            pltpu.VMEM((_M,), jnp.int32),
            pltpu.VMEM((_SPW * _N,), jnp.int32),
            pltpu.VMEM((_SPW,), jnp.int32),
            pltpu.VMEM((_SPW,), jnp.int32),
            pltpu.VMEM((_SPW, _N), jnp.float32),
            pltpu.VMEM((2 * _N,), jnp.float32),
            pltpu.VMEM((2 * _M,), jnp.float32),
            pltpu.SemaphoreType.DMA,
        ],
    )(_sc_gather)
    return sc(Xf, rank, tri, base).reshape(B, _M)
